# Initial kernel scaffold; baseline (speedup 1.0000x reference)
import jax
import jax.numpy as jnp
from jax.experimental import pallas as pl


def kernel(x, edge_index, batch, vn_w, w1, b1, g1, be1, w2, b2, g2, be2):
    raise NotImplementedError("write your pallas kernel here")


# speedup vs baseline: 11.9957x; 11.9957x over previous
import jax
import jax.numpy as jnp
from jax.experimental import pallas as pl
from jax.experimental.pallas import tpu as pltpu

_N, _D, _G = 50000, 256, 128
_BLK = 2000
_NB = _N // _BLK


def _fused(batch_ref, x_ref, vn_ref, w1_ref, bias1_ref, w2_ref, bias2_ref,
           h_ref, t_ref, acc_ref):
    i = pl.program_id(0)
    vn = vn_ref[0, :]
    hb = x_ref[...] + vn[None, :]
    h_ref[...] = hb
    ids = batch_ref[0, 0, :]
    oh = (jax.lax.broadcasted_iota(jnp.int32, (_G, _BLK), 0)
          == ids[None, :]).astype(jnp.float32)
    part = jnp.dot(oh, hb, preferred_element_type=jnp.float32)

    @pl.when(i == 0)
    def _init():
        acc_ref[...] = part

    @pl.when(i > 0)
    def _accum():
        acc_ref[...] += part

    @pl.when(i == _NB - 1)
    def _finish():
        pooled = acc_ref[...] + vn[None, :]
        t = jnp.dot(pooled, w1_ref[...], preferred_element_type=jnp.float32)
        t = jnp.maximum(t + bias1_ref[0, :], 0.0)
        t = jnp.dot(t, w2_ref[...], preferred_element_type=jnp.float32)
        t_ref[...] = jnp.maximum(t + bias2_ref[0, :], 0.0)


def kernel(x, edge_index, batch, vn_w, w1, b1, g1, be1, w2, b2, g2, be2):
    del edge_index
    eps = 1e-5
    inv = 1.0 / jnp.sqrt(1.0 + eps)
    s1 = g1 * inv
    w1s = w1 * s1[None, :]
    bias1 = (b1 * s1 + be1).reshape(1, 2 * _D)
    s2 = g2 * inv
    w2s = w2 * s2[None, :]
    bias2 = (b2 * s2 + be2).reshape(1, _D)
    batch3 = batch.reshape(_NB, 1, _BLK)

    h, t = pl.pallas_call(
        _fused,
        grid=(_NB,),
        in_specs=[
            pl.BlockSpec((1, 1, _BLK), lambda i: (i, 0, 0)),
            pl.BlockSpec((_BLK, _D), lambda i: (i, 0)),
            pl.BlockSpec((1, _D), lambda i: (0, 0)),
            pl.BlockSpec((_D, 2 * _D), lambda i: (0, 0)),
            pl.BlockSpec((1, 2 * _D), lambda i: (0, 0)),
            pl.BlockSpec((2 * _D, _D), lambda i: (0, 0)),
            pl.BlockSpec((1, _D), lambda i: (0, 0)),
        ],
        out_specs=[
            pl.BlockSpec((_BLK, _D), lambda i: (i, 0)),
            pl.BlockSpec((_G, _D), lambda i: (0, 0)),
        ],
        out_shape=[
            jax.ShapeDtypeStruct((_N, _D), jnp.float32),
            jax.ShapeDtypeStruct((_G, _D), jnp.float32),
        ],
        scratch_shapes=[pltpu.VMEM((_G, _D), jnp.float32)],
        compiler_params=pltpu.CompilerParams(
            dimension_semantics=("arbitrary",),
        ),
    )(batch3, x, vn_w, w1s, bias1, w2s, bias2)
    return (h, t)


# speedup vs baseline: 13.4643x; 1.1224x over previous
import jax
import jax.numpy as jnp
from jax.experimental import pallas as pl
from jax.experimental.pallas import tpu as pltpu

_N, _D, _G = 50000, 256, 128
_BLK = 5000
_NB = _N // _BLK


def _fused(batch_ref, x_ref, vn_ref, w1_ref, bias1_ref, w2_ref, bias2_ref,
           h_ref, t_ref, acc_ref):
    i = pl.program_id(0)
    vn = vn_ref[0, :]
    hb = x_ref[...] + vn[None, :]
    h_ref[...] = hb
    ids = batch_ref[0, 0, :]
    oh = (jax.lax.broadcasted_iota(jnp.int32, (_G, _BLK), 0)
          == ids[None, :]).astype(jnp.float32)
    part = jnp.dot(oh, hb, preferred_element_type=jnp.float32)

    @pl.when(i == 0)
    def _init():
        acc_ref[...] = part

    @pl.when(i > 0)
    def _accum():
        acc_ref[...] += part

    @pl.when(i == _NB - 1)
    def _finish():
        pooled = acc_ref[...] + vn[None, :]
        t = jnp.dot(pooled, w1_ref[...], preferred_element_type=jnp.float32)
        t = jnp.maximum(t + bias1_ref[0, :], 0.0)
        t = jnp.dot(t, w2_ref[...], preferred_element_type=jnp.float32)
        t_ref[...] = jnp.maximum(t + bias2_ref[0, :], 0.0)


def kernel(x, edge_index, batch, vn_w, w1, b1, g1, be1, w2, b2, g2, be2):
    del edge_index
    eps = 1e-5
    inv = 1.0 / jnp.sqrt(1.0 + eps)
    s1 = g1 * inv
    w1s = w1 * s1[None, :]
    bias1 = (b1 * s1 + be1).reshape(1, 2 * _D)
    s2 = g2 * inv
    w2s = w2 * s2[None, :]
    bias2 = (b2 * s2 + be2).reshape(1, _D)
    batch3 = batch.reshape(_NB, 1, _BLK)

    h, t = pl.pallas_call(
        _fused,
        grid=(_NB,),
        in_specs=[
            pl.BlockSpec((1, 1, _BLK), lambda i: (i, 0, 0)),
            pl.BlockSpec((_BLK, _D), lambda i: (i, 0)),
            pl.BlockSpec((1, _D), lambda i: (0, 0)),
            pl.BlockSpec((_D, 2 * _D), lambda i: (0, 0)),
            pl.BlockSpec((1, 2 * _D), lambda i: (0, 0)),
            pl.BlockSpec((2 * _D, _D), lambda i: (0, 0)),
            pl.BlockSpec((1, _D), lambda i: (0, 0)),
        ],
        out_specs=[
            pl.BlockSpec((_BLK, _D), lambda i: (i, 0)),
            pl.BlockSpec((_G, _D), lambda i: (0, 0)),
        ],
        out_shape=[
            jax.ShapeDtypeStruct((_N, _D), jnp.float32),
            jax.ShapeDtypeStruct((_G, _D), jnp.float32),
        ],
        scratch_shapes=[pltpu.VMEM((_G, _D), jnp.float32)],
        compiler_params=pltpu.CompilerParams(
            dimension_semantics=("arbitrary",),
        ),
    )(batch3, x, vn_w, w1s, bias1, w2s, bias2)
    return (h, t)


# speedup vs baseline: 14.0302x; 1.0420x over previous
import jax
import jax.numpy as jnp
from jax.experimental import pallas as pl
from jax.experimental.pallas import tpu as pltpu

_N, _D, _G = 50000, 256, 128
_BLK = 10000
_NB = _N // _BLK


def _fused(batch_ref, x_ref, vn_ref, w1_ref, bias1_ref, w2_ref, bias2_ref,
           h_ref, t_ref, acc_ref):
    i = pl.program_id(0)
    vn = vn_ref[0, :]
    hb = x_ref[...] + vn[None, :]
    h_ref[...] = hb
    ids = batch_ref[0, 0, :]
    oh = (jax.lax.broadcasted_iota(jnp.int32, (_G, _BLK), 0)
          == ids[None, :]).astype(jnp.float32)
    part = jnp.dot(oh, hb, preferred_element_type=jnp.float32)

    @pl.when(i == 0)
    def _init():
        acc_ref[...] = part

    @pl.when(i > 0)
    def _accum():
        acc_ref[...] += part

    @pl.when(i == _NB - 1)
    def _finish():
        pooled = acc_ref[...] + vn[None, :]
        t = jnp.dot(pooled, w1_ref[...], preferred_element_type=jnp.float32)
        t = jnp.maximum(t + bias1_ref[0, :], 0.0)
        t = jnp.dot(t, w2_ref[...], preferred_element_type=jnp.float32)
        t_ref[...] = jnp.maximum(t + bias2_ref[0, :], 0.0)


def kernel(x, edge_index, batch, vn_w, w1, b1, g1, be1, w2, b2, g2, be2):
    del edge_index
    eps = 1e-5
    inv = 1.0 / jnp.sqrt(1.0 + eps)
    s1 = g1 * inv
    w1s = w1 * s1[None, :]
    bias1 = (b1 * s1 + be1).reshape(1, 2 * _D)
    s2 = g2 * inv
    w2s = w2 * s2[None, :]
    bias2 = (b2 * s2 + be2).reshape(1, _D)
    batch3 = batch.reshape(_NB, 1, _BLK)

    h, t = pl.pallas_call(
        _fused,
        grid=(_NB,),
        in_specs=[
            pl.BlockSpec((1, 1, _BLK), lambda i: (i, 0, 0)),
            pl.BlockSpec((_BLK, _D), lambda i: (i, 0)),
            pl.BlockSpec((1, _D), lambda i: (0, 0)),
            pl.BlockSpec((_D, 2 * _D), lambda i: (0, 0)),
            pl.BlockSpec((1, 2 * _D), lambda i: (0, 0)),
            pl.BlockSpec((2 * _D, _D), lambda i: (0, 0)),
            pl.BlockSpec((1, _D), lambda i: (0, 0)),
        ],
        out_specs=[
            pl.BlockSpec((_BLK, _D), lambda i: (i, 0)),
            pl.BlockSpec((_G, _D), lambda i: (0, 0)),
        ],
        out_shape=[
            jax.ShapeDtypeStruct((_N, _D), jnp.float32),
            jax.ShapeDtypeStruct((_G, _D), jnp.float32),
        ],
        scratch_shapes=[pltpu.VMEM((_G, _D), jnp.float32)],
        compiler_params=pltpu.CompilerParams(
            dimension_semantics=("arbitrary",),
        ),
    )(batch3, x, vn_w, w1s, bias1, w2s, bias2)
    return (h, t)


# speedup vs baseline: 14.6095x; 1.0413x over previous
import jax
import jax.numpy as jnp
from jax.experimental import pallas as pl
from jax.experimental.pallas import tpu as pltpu

_N, _D, _G = 50000, 256, 128
_BLK = 10000
_NB = _N // _BLK


def _stream(batch_ref, x_ref, vn_ref, h_ref, part_ref):
    vn = vn_ref[0, :]
    hb = x_ref[...] + vn[None, :]
    h_ref[...] = hb
    ids = batch_ref[0, 0, :]
    oh = (jax.lax.broadcasted_iota(jnp.int32, (_G, _BLK), 0)
          == ids[None, :]).astype(jnp.float32)
    part_ref[0] = jnp.dot(oh, hb, preferred_element_type=jnp.float32)


def _finalize(part_ref, vn_ref, w1_ref, bias1_ref, w2_ref, bias2_ref, t_ref):
    pooled = jnp.sum(part_ref[...], axis=0) + vn_ref[0, :][None, :]
    t = jnp.dot(pooled, w1_ref[...], preferred_element_type=jnp.float32)
    t = jnp.maximum(t + bias1_ref[0, :], 0.0)
    t = jnp.dot(t, w2_ref[...], preferred_element_type=jnp.float32)
    t_ref[...] = jnp.maximum(t + bias2_ref[0, :], 0.0)


def kernel(x, edge_index, batch, vn_w, w1, b1, g1, be1, w2, b2, g2, be2):
    del edge_index
    eps = 1e-5
    inv = 1.0 / jnp.sqrt(1.0 + eps)
    s1 = g1 * inv
    w1s = w1 * s1[None, :]
    bias1 = (b1 * s1 + be1).reshape(1, 2 * _D)
    s2 = g2 * inv
    w2s = w2 * s2[None, :]
    bias2 = (b2 * s2 + be2).reshape(1, _D)
    batch3 = batch.reshape(_NB, 1, _BLK)

    h, parts = pl.pallas_call(
        _stream,
        grid=(_NB,),
        in_specs=[
            pl.BlockSpec((1, 1, _BLK), lambda i: (i, 0, 0)),
            pl.BlockSpec((_BLK, _D), lambda i: (i, 0)),
            pl.BlockSpec((1, _D), lambda i: (0, 0)),
        ],
        out_specs=[
            pl.BlockSpec((_BLK, _D), lambda i: (i, 0)),
            pl.BlockSpec((1, _G, _D), lambda i: (i, 0, 0)),
        ],
        out_shape=[
            jax.ShapeDtypeStruct((_N, _D), jnp.float32),
            jax.ShapeDtypeStruct((_NB, _G, _D), jnp.float32),
        ],
        compiler_params=pltpu.CompilerParams(
            dimension_semantics=("parallel",),
        ),
    )(batch3, x, vn_w)

    t = pl.pallas_call(
        _finalize,
        out_shape=jax.ShapeDtypeStruct((_G, _D), jnp.float32),
    )(parts, vn_w, w1s, bias1, w2s, bias2)
    return (h, t)
